# baseline (speedup 1.0000x reference)
import jax
import jax.numpy as jnp
from jax import lax
from jax.experimental import pallas as pl
from jax.experimental.pallas import tpu as pltpu
from jax.experimental.pallas import tpu_sc as plsc

B = 16384
OUT_D = 149
NC = 2
NS = 16
NW = NC * NS
ROWS_PER_W = B // NW
CHUNK = 128
NCHUNK = ROWS_PER_W // CHUNK
NGROUP = CHUNK // 16

COL_UNIT = 0
COL_NUM = 64
COL_AB = 75
COL_TR = 91
COL_ST = 107
COL_RES = 123
COL_DEF = 129
COL_MOV = 139


def _full(v):
    return jnp.full((16,), v, jnp.int32)


def _prep_exp_table(tab_v, q_v, e_v):
    lanes = lax.iota(jnp.int32, 16)
    s = jnp.zeros((16,), jnp.float32)
    for d in range(16):
        s = s + (plsc.load_gather(tab_v, [lanes, _full(d)])
                 * plsc.load_gather(q_v, [_full(d)]))
    e_v[...] = s
    s0 = plsc.load_gather(e_v, [_full(0)])
    e_v[...] = jnp.exp(s - s0)


def _body(uids, ab_i, tr_i, st_i, num, res, dfs, mov,
          utab, atab, ttab, stab, qa, qt, qs,
          out,
          uids_v, ab_v, tr_v, st_v, num_v, res_v, def_v, mov_v,
          rows_v, out_v, atab_v, ttab_v, stab_v, ea_v, et_v, es_v,
          qa_v, qt_v, qs_v, sem_in, sem_g, sem_out):
    wid = lax.axis_index("s") * NC + lax.axis_index("c")
    base_w = wid * ROWS_PER_W

    pltpu.sync_copy(atab, atab_v.at[pl.ds(0, 14)])
    pltpu.sync_copy(ttab, ttab_v.at[pl.ds(0, 12)])
    pltpu.sync_copy(stab, stab_v.at[pl.ds(0, 4)])
    pltpu.sync_copy(qa, qa_v)
    pltpu.sync_copy(qt, qt_v)
    pltpu.sync_copy(qs, qs_v)
    _prep_exp_table(atab_v, qa_v, ea_v)
    _prep_exp_table(ttab_v, qt_v, et_v)
    _prep_exp_table(stab_v, qs_v, es_v)

    def stage(c, b):
        base = base_w + c * CHUNK
        sl = pl.ds(base, CHUNK)
        mk = pltpu.async_copy
        return [
            mk(uids.at[sl], uids_v.at[b], sem_in.at[b]),
            mk(ab_i.at[sl], ab_v.at[b], sem_in.at[b]),
            mk(tr_i.at[sl], tr_v.at[b], sem_in.at[b]),
            mk(st_i.at[sl], st_v.at[b], sem_in.at[b]),
            mk(num.at[sl], num_v.at[b], sem_in.at[b]),
            mk(res.at[sl], res_v.at[b], sem_in.at[b]),
            mk(dfs.at[sl], def_v.at[b], sem_in.at[b]),
            mk(mov.at[sl], mov_v.at[b], sem_in.at[b]),
        ]

    def attend(idx_v, n_l, tab_v, e_v, out_col, rowid, outb):
        idxs = [plsc.load_gather(idx_v, [rowid, _full(l)]) for l in range(n_l)]
        es = [plsc.load_gather(e_v, [ix]) for ix in idxs]
        denom = es[0]
        for e in es[1:]:
            denom = denom + e
        inv = 1.0 / denom
        ws = [e * inv for e in es]
        for d in range(16):
            cold = _full(d)
            acc = ws[0] * plsc.load_gather(tab_v, [idxs[0], cold])
            for l in range(1, n_l):
                acc = acc + ws[l] * plsc.load_gather(tab_v, [idxs[l], cold])
            plsc.store_scatter(outb, [rowid, _full(out_col + d)], acc)

    def copy_cols(src_v, n_d, out_col, rowid, outb, scale=None):
        for d in range(n_d):
            v = plsc.load_gather(src_v, [rowid, _full(d)])
            if scale is not None:
                v = v * scale
            plsc.store_scatter(outb, [rowid, _full(out_col + d)], v)

    in_descs = {0: stage(0, 0)}
    g_descs = {}
    out_descs = {}
    for c in range(NCHUNK):
        b = c % 2
        base = base_w + c * CHUNK
        for d in in_descs.pop(c):
            d.wait()
        g_descs[c] = pltpu.async_copy(utab.at[uids_v.at[b]],
                                      rows_v.at[b], sem_g.at[b])
        if c + 1 < NCHUNK:
            in_descs[c + 1] = stage(c + 1, 1 - b)
        if c - 2 >= 0:
            out_descs.pop(c - 2).wait()

        numb, resb, defb, movb = (num_v.at[b], res_v.at[b],
                                  def_v.at[b], mov_v.at[b])
        abb, trb, stb = ab_v.at[b], tr_v.at[b], st_v.at[b]
        outb, rowsb = out_v.at[b], rows_v.at[b]

        def group_ac(g, carry):
            rowid = lax.iota(jnp.int32, 16) + g * 16
            copy_cols(numb, 11, COL_NUM, rowid, outb)
            copy_cols(resb, 6, COL_RES, rowid, outb)
            copy_cols(defb, 10, COL_DEF, rowid, outb)
            copy_cols(movb, 10, COL_MOV, rowid, outb, scale=0.1)
            attend(abb, 4, atab_v, ea_v, COL_AB, rowid, outb)
            attend(trb, 3, ttab_v, et_v, COL_TR, rowid, outb)
            attend(stb, 2, stab_v, es_v, COL_ST, rowid, outb)
            return carry

        lax.fori_loop(0, NGROUP, group_ac, 0)
        g_descs.pop(c).wait()

        def group_b(g, carry):
            rbase = g * 16
            for j in range(16):
                r = rbase + j
                for k in range(4):
                    outb[r, pl.ds(k * 16, 16)] = rowsb[r, pl.ds(k * 16, 16)]
            return carry

        lax.fori_loop(0, NGROUP, group_b, 0)
        out_descs[c] = pltpu.async_copy(
            out_v.at[b], out.at[pl.ds(base, CHUNK)], sem_out.at[b])
    for c in sorted(out_descs):
        out_descs.pop(c).wait()


def kernel(unit_type_ids, ability_indices, trait_indices, status_indices,
           numerical, resistances, defenses, movement_costs,
           unit_type_table, ability_table, trait_table, status_table,
           ability_query, trait_query, status_query):
    mesh = plsc.VectorSubcoreMesh(core_axis_name="c", subcore_axis_name="s")
    f32 = jnp.float32
    i32 = jnp.int32
    kfn = pl.kernel(
        _body,
        mesh=mesh,
        compiler_params=pltpu.CompilerParams(
            use_tc_tiling_on_sc=False, needs_layout_passes=False),
        out_type=jax.ShapeDtypeStruct((B, OUT_D), f32),
        scratch_types=[
            pltpu.VMEM((2, CHUNK), i32),
            pltpu.VMEM((2, CHUNK, 4), i32),
            pltpu.VMEM((2, CHUNK, 3), i32),
            pltpu.VMEM((2, CHUNK, 2), i32),
            pltpu.VMEM((2, CHUNK, 11), f32),
            pltpu.VMEM((2, CHUNK, 6), f32),
            pltpu.VMEM((2, CHUNK, 10), f32),
            pltpu.VMEM((2, CHUNK, 10), f32),
            pltpu.VMEM((2, CHUNK, 64), f32),
            pltpu.VMEM((2, CHUNK, OUT_D), f32),
            pltpu.VMEM((16, 16), f32),
            pltpu.VMEM((16, 16), f32),
            pltpu.VMEM((16, 16), f32),
            pltpu.VMEM((16,), f32),
            pltpu.VMEM((16,), f32),
            pltpu.VMEM((16,), f32),
            pltpu.VMEM((16,), f32),
            pltpu.VMEM((16,), f32),
            pltpu.VMEM((16,), f32),
            pltpu.SemaphoreType.DMA((2,)),
            pltpu.SemaphoreType.DMA((2,)),
            pltpu.SemaphoreType.DMA((2,)),
        ],
    )
    return kfn(
        unit_type_ids.astype(i32),
        ability_indices.astype(i32),
        trait_indices.astype(i32),
        status_indices.astype(i32),
        numerical, resistances, defenses, movement_costs,
        unit_type_table, ability_table, trait_table, status_table,
        ability_query, trait_query, status_query,
    )


# speedup vs baseline: 1.5004x; 1.5004x over previous
import jax
import jax.numpy as jnp
from jax import lax
from jax.experimental import pallas as pl
from jax.experimental.pallas import tpu as pltpu
from jax.experimental.pallas import tpu_sc as plsc

B = 16384
OUT_D = 149
NC = 2
NS = 16
NW = NC * NS
ROWS_PER_W = B // NW
CHUNK = 128
NCHUNK = ROWS_PER_W // CHUNK
NGROUP = CHUNK // 16

COL_UNIT = 0
COL_NUM = 64
COL_AB = 75
COL_TR = 91
COL_ST = 107
COL_RES = 123
COL_DEF = 129
COL_MOV = 139

F_AB = 0
F_TR = 4
F_ST = 7
F_NUM = 9
F_RES = 20
F_DEF = 26
F_MOV = 36
F_W = 46

OUT_W = CHUNK * OUT_D


def _full(v):
    return jnp.full((16,), v, jnp.int32)


def _prep_exp_table(tab_v, q_v, e_v):
    lanes = lax.iota(jnp.int32, 16)
    s = jnp.zeros((16,), jnp.float32)
    for d in range(16):
        s = s + (plsc.load_gather(tab_v, [lanes, _full(d)])
                 * plsc.load_gather(q_v, [_full(d)]))
    e_v[...] = s
    s0 = plsc.load_gather(e_v, [_full(0)])
    e_v[...] = jnp.exp(s - s0)


def _body(uids, feat, utab, atab, ttab, stab, qa, qt, qs,
          out,
          uids_v, feat_v, rows_v, out_v,
          atab_v, ttab_v, stab_v, ea_v, et_v, es_v,
          qa_v, qt_v, qs_v, sem_in, sem_g, sem_out):
    wid = lax.axis_index("s") * NC + lax.axis_index("c")
    base_w = wid * ROWS_PER_W

    pltpu.sync_copy(atab, atab_v.at[pl.ds(0, 14)])
    pltpu.sync_copy(ttab, ttab_v.at[pl.ds(0, 12)])
    pltpu.sync_copy(stab, stab_v.at[pl.ds(0, 4)])
    pltpu.sync_copy(qa, qa_v)
    pltpu.sync_copy(qt, qt_v)
    pltpu.sync_copy(qs, qs_v)
    _prep_exp_table(atab_v, qa_v, ea_v)
    _prep_exp_table(ttab_v, qt_v, et_v)
    _prep_exp_table(stab_v, qs_v, es_v)

    def stage(c, b):
        base = base_w + c * CHUNK
        mk = pltpu.async_copy
        return [
            mk(uids.at[pl.ds(base, CHUNK)], uids_v.at[b], sem_in.at[b]),
            mk(feat.at[pl.ds(base * F_W, CHUNK * F_W)],
               feat_v.at[b, pl.ds(0, CHUNK * F_W)], sem_in.at[b]),
        ]

    def attend(featb, f_col, n_l, tab_v, e_v, out_col, rowf, rowoff, outb):
        idxs = [plsc.bitcast(
            plsc.load_gather(featb, [rowf + _full(f_col + l)]), jnp.int32)
            for l in range(n_l)]
        es = [plsc.load_gather(e_v, [ix]) for ix in idxs]
        denom = es[0]
        for e in es[1:]:
            denom = denom + e
        inv = 1.0 / denom
        ws = [e * inv for e in es]
        for d in range(16):
            acc = ws[0] * plsc.load_gather(tab_v, [idxs[0], _full(d)])
            for l in range(1, n_l):
                acc = acc + ws[l] * plsc.load_gather(tab_v, [idxs[l], _full(d)])
            plsc.store_scatter(outb, [rowoff + _full(out_col + d)], acc)

    in_descs = {0: stage(0, 0)}
    g_descs = {}
    out_descs = {}
    for c in range(NCHUNK):
        b = c % 2
        base = base_w + c * CHUNK
        for d in in_descs.pop(c):
            d.wait()
        g_descs[c] = pltpu.async_copy(utab.at[uids_v.at[b]],
                                      rows_v.at[b], sem_g.at[b])
        if c + 1 < NCHUNK:
            in_descs[c + 1] = stage(c + 1, 1 - b)
        if c - 2 >= 0:
            out_descs.pop(c - 2).wait()

        featb = feat_v.at[b]
        outb, rowsb = out_v.at[b], rows_v.at[b]

        def group_ac(g, carry):
            rbase = g * 16
            for j in range(16):
                r = rbase + j
                roff = r * OUT_D
                foff = r * F_W
                outb[pl.ds(roff + COL_NUM, 16)] = featb[pl.ds(foff + F_NUM, 16)]
                outb[pl.ds(roff + COL_RES, 16)] = featb[pl.ds(foff + F_RES, 16)]
                outb[pl.ds(roff + COL_DEF, 16)] = featb[pl.ds(foff + F_DEF, 16)]
                outb[pl.ds(roff + COL_MOV, 16)] = featb[pl.ds(foff + F_MOV, 16)] * 0.1
            rowid = lax.iota(jnp.int32, 16) + rbase
            rowf = rowid * F_W
            rowoff = rowid * OUT_D
            attend(featb, F_AB, 4, atab_v, ea_v, COL_AB, rowf, rowoff, outb)
            attend(featb, F_TR, 3, ttab_v, et_v, COL_TR, rowf, rowoff, outb)
            attend(featb, F_ST, 2, stab_v, es_v, COL_ST, rowf, rowoff, outb)
            return carry

        lax.fori_loop(0, NGROUP, group_ac, 0)
        g_descs.pop(c).wait()

        def group_b(g, carry):
            rbase = g * 16
            for j in range(16):
                r = rbase + j
                roff = r * OUT_D
                for k in range(4):
                    outb[pl.ds(roff + k * 16, 16)] = rowsb[r, pl.ds(k * 16, 16)]
            return carry

        lax.fori_loop(0, NGROUP, group_b, 0)
        out_descs[c] = pltpu.async_copy(
            out_v.at[b, pl.ds(0, OUT_W)],
            out.at[pl.ds(base * OUT_D, OUT_W)], sem_out.at[b])
    for c in sorted(out_descs):
        out_descs.pop(c).wait()


def kernel(unit_type_ids, ability_indices, trait_indices, status_indices,
           numerical, resistances, defenses, movement_costs,
           unit_type_table, ability_table, trait_table, status_table,
           ability_query, trait_query, status_query):
    mesh = plsc.VectorSubcoreMesh(core_axis_name="c", subcore_axis_name="s")
    f32 = jnp.float32
    i32 = jnp.int32
    bc = lambda x: lax.bitcast_convert_type(x.astype(i32), f32)
    feat = jnp.concatenate(
        [bc(ability_indices), bc(trait_indices), bc(status_indices),
         numerical, resistances, defenses, movement_costs],
        axis=1).reshape(-1)
    kfn = pl.kernel(
        _body,
        mesh=mesh,
        compiler_params=pltpu.CompilerParams(
            use_tc_tiling_on_sc=False, needs_layout_passes=False),
        out_type=jax.ShapeDtypeStruct((B * OUT_D,), f32),
        scratch_types=[
            pltpu.VMEM((2, CHUNK), i32),
            pltpu.VMEM((2, CHUNK * F_W + 16), f32),
            pltpu.VMEM((2, CHUNK, 64), f32),
            pltpu.VMEM((2, OUT_W + 16), f32),
            pltpu.VMEM((16, 16), f32),
            pltpu.VMEM((16, 16), f32),
            pltpu.VMEM((16, 16), f32),
            pltpu.VMEM((16,), f32),
            pltpu.VMEM((16,), f32),
            pltpu.VMEM((16,), f32),
            pltpu.VMEM((16,), f32),
            pltpu.VMEM((16,), f32),
            pltpu.VMEM((16,), f32),
            pltpu.SemaphoreType.DMA((2,)),
            pltpu.SemaphoreType.DMA((2,)),
            pltpu.SemaphoreType.DMA((2,)),
        ],
    )
    out_flat = kfn(
        unit_type_ids.astype(i32), feat,
        unit_type_table, ability_table, trait_table, status_table,
        ability_query, trait_query, status_query,
    )
    return out_flat.reshape(B, OUT_D)


# speedup vs baseline: 1.5018x; 1.0009x over previous
import jax
import jax.numpy as jnp
from jax import lax
from jax.experimental import pallas as pl
from jax.experimental.pallas import tpu as pltpu
from jax.experimental.pallas import tpu_sc as plsc

B = 16384
OUT_D = 149
NC = 2
NS = 16
NW = NC * NS
ROWS_PER_W = B // NW
CHUNK = 128
NCHUNK = ROWS_PER_W // CHUNK
NGROUP = CHUNK // 16

COL_UNIT = 0
COL_NUM = 64
COL_AB = 75
COL_TR = 91
COL_ST = 107
COL_RES = 123
COL_DEF = 129
COL_MOV = 139

F_AB = 0
F_TR = 4
F_ST = 7
F_NUM = 9
F_RES = 20
F_DEF = 26
F_MOV = 36
F_W = 46

OUT_W = CHUNK * OUT_D


def _full(v):
    return jnp.full((16,), v, jnp.int32)


def _prep_exp_table(tab_v, q_v, e_v):
    lanes = lax.iota(jnp.int32, 16)
    s = jnp.zeros((16,), jnp.float32)
    for d in range(16):
        s = s + (plsc.load_gather(tab_v, [lanes, _full(d)])
                 * plsc.load_gather(q_v, [_full(d)]))
    e_v[...] = s
    s0 = plsc.load_gather(e_v, [_full(0)])
    e_v[...] = jnp.exp(s - s0)


def _body(uids, feat, utab, atab, ttab, stab, qa, qt, qs,
          out,
          uids_v, feat_v, rows_v, out_v,
          atab_v, ttab_v, stab_v, ea_v, et_v, es_v,
          qa_v, qt_v, qs_v, sem_in, sem_g, sem_out):
    wid = lax.axis_index("s") * NC + lax.axis_index("c")
    base_w = wid * ROWS_PER_W

    pltpu.sync_copy(atab, atab_v.at[pl.ds(0, 14)])
    pltpu.sync_copy(ttab, ttab_v.at[pl.ds(0, 12)])
    pltpu.sync_copy(stab, stab_v.at[pl.ds(0, 4)])
    pltpu.sync_copy(qa, qa_v)
    pltpu.sync_copy(qt, qt_v)
    pltpu.sync_copy(qs, qs_v)
    _prep_exp_table(atab_v, qa_v, ea_v)
    _prep_exp_table(ttab_v, qt_v, et_v)
    _prep_exp_table(stab_v, qs_v, es_v)

    def stage(c, b):
        base = base_w + c * CHUNK
        mk = pltpu.async_copy
        return [
            mk(uids.at[pl.ds(base, CHUNK)], uids_v.at[b], sem_in.at[b]),
            mk(feat.at[pl.ds(base * F_W, CHUNK * F_W)],
               feat_v.at[b, pl.ds(0, CHUNK * F_W)], sem_in.at[b]),
        ]

    def attend(featb, f_col, n_l, tab_v, e_v, out_col, rowf, rowoff, outb):
        idxs = [plsc.bitcast(
            plsc.load_gather(featb, [rowf + _full(f_col + l)]), jnp.int32)
            for l in range(n_l)]
        es = [plsc.load_gather(e_v, [ix]) for ix in idxs]
        denom = es[0]
        for e in es[1:]:
            denom = denom + e
        inv = 1.0 / denom
        ws = [e * inv for e in es]
        for d in range(16):
            acc = ws[0] * plsc.load_gather(tab_v, [idxs[0], _full(d)])
            for l in range(1, n_l):
                acc = acc + ws[l] * plsc.load_gather(tab_v, [idxs[l], _full(d)])
            plsc.store_scatter(outb, [rowoff + _full(out_col + d)], acc)

    in_descs = {0: stage(0, 0)}
    g_descs = {}
    out_descs = {}
    for c in range(NCHUNK):
        b = c % 2
        base = base_w + c * CHUNK
        with jax.named_scope("wait_in"):
            for d in in_descs.pop(c):
                d.wait()
        g_descs[c] = pltpu.async_copy(utab.at[uids_v.at[b]],
                                      rows_v.at[b], sem_g.at[b])
        if c + 1 < NCHUNK:
            in_descs[c + 1] = stage(c + 1, 1 - b)
        if c - 2 >= 0:
            out_descs.pop(c - 2).wait()

        featb = feat_v.at[b]
        outb, rowsb = out_v.at[b], rows_v.at[b]

        def group_ac(g, carry):
            rbase = g * 16
            for j in range(16):
                r = rbase + j
                roff = r * OUT_D
                foff = r * F_W
                outb[pl.ds(roff + COL_NUM, 16)] = featb[pl.ds(foff + F_NUM, 16)]
                outb[pl.ds(roff + COL_RES, 16)] = featb[pl.ds(foff + F_RES, 16)]
                outb[pl.ds(roff + COL_DEF, 16)] = featb[pl.ds(foff + F_DEF, 16)]
                outb[pl.ds(roff + COL_MOV, 16)] = featb[pl.ds(foff + F_MOV, 16)] * 0.1
            rowid = lax.iota(jnp.int32, 16) + rbase
            rowf = rowid * F_W
            rowoff = rowid * OUT_D
            attend(featb, F_AB, 4, atab_v, ea_v, COL_AB, rowf, rowoff, outb)
            attend(featb, F_TR, 3, ttab_v, et_v, COL_TR, rowf, rowoff, outb)
            attend(featb, F_ST, 2, stab_v, es_v, COL_ST, rowf, rowoff, outb)
            return carry

        with jax.named_scope("phase_ac"):
            lax.fori_loop(0, NGROUP, group_ac, 0)
        with jax.named_scope("wait_gather"):
            g_descs.pop(c).wait()

        def group_b(g, carry):
            rbase = g * 16
            for j in range(16):
                r = rbase + j
                roff = r * OUT_D
                for k in range(4):
                    outb[pl.ds(roff + k * 16, 16)] = rowsb[r, pl.ds(k * 16, 16)]
            return carry

        with jax.named_scope("phase_b"):
            lax.fori_loop(0, NGROUP, group_b, 0)
        out_descs[c] = pltpu.async_copy(
            out_v.at[b, pl.ds(0, OUT_W)],
            out.at[pl.ds(base * OUT_D, OUT_W)], sem_out.at[b])
    for c in sorted(out_descs):
        out_descs.pop(c).wait()


def kernel(unit_type_ids, ability_indices, trait_indices, status_indices,
           numerical, resistances, defenses, movement_costs,
           unit_type_table, ability_table, trait_table, status_table,
           ability_query, trait_query, status_query):
    mesh = plsc.VectorSubcoreMesh(core_axis_name="c", subcore_axis_name="s")
    f32 = jnp.float32
    i32 = jnp.int32
    bc = lambda x: lax.bitcast_convert_type(x.astype(i32), f32)
    feat = jnp.concatenate(
        [bc(ability_indices), bc(trait_indices), bc(status_indices),
         numerical, resistances, defenses, movement_costs],
        axis=1).reshape(-1)
    kfn = pl.kernel(
        _body,
        mesh=mesh,
        compiler_params=pltpu.CompilerParams(
            use_tc_tiling_on_sc=False, needs_layout_passes=False),
        out_type=jax.ShapeDtypeStruct((B * OUT_D,), f32),
        scratch_types=[
            pltpu.VMEM((2, CHUNK), i32),
            pltpu.VMEM((2, CHUNK * F_W + 16), f32),
            pltpu.VMEM((2, CHUNK, 64), f32),
            pltpu.VMEM((2, OUT_W + 16), f32),
            pltpu.VMEM((16, 16), f32),
            pltpu.VMEM((16, 16), f32),
            pltpu.VMEM((16, 16), f32),
            pltpu.VMEM((16,), f32),
            pltpu.VMEM((16,), f32),
            pltpu.VMEM((16,), f32),
            pltpu.VMEM((16,), f32),
            pltpu.VMEM((16,), f32),
            pltpu.VMEM((16,), f32),
            pltpu.SemaphoreType.DMA((2,)),
            pltpu.SemaphoreType.DMA((2,)),
            pltpu.SemaphoreType.DMA((2,)),
        ],
    )
    out_flat = kfn(
        unit_type_ids.astype(i32), feat,
        unit_type_table, ability_table, trait_table, status_table,
        ability_query, trait_query, status_query,
    )
    return out_flat.reshape(B, OUT_D)
